# baseline (speedup 1.0000x reference)
import jax
import jax.numpy as jnp
from jax import lax
from jax.experimental import pallas as pl
from jax.experimental.pallas import tpu as pltpu

B = 4
SEQ = 8192
D_MODEL = 4096
HIDDEN = 4096
NUM_LAYERS = 16

NSTREAM = 8
R_BLK = 128
ROWS_PER_STREAM = B * SEQ // NSTREAM
NS = ROWS_PER_STREAM // R_BLK

H_BLK = 256
NSUB1 = 4
SUB1 = H_BLK // NSUB1
NSUB2 = 2
SUB2 = H_BLK // NSUB2
NH = HIDDEN // H_BLK
GRID = NS + NH + 1


def _router_kernel(*refs):
    x_refs = refs[:NSTREAM]
    (w1a_ref, w1b_ref, w1c_ref, w1d_ref,
     w2a_ref, w2b_ref,
     b1_ref, b2_ref, w3_ref, b3_ref,
     logits_ref, idx_ref, acc8_ref, xp_ref, h2_ref) = refs[NSTREAM:]
    w1_refs = (w1a_ref, w1b_ref, w1c_ref, w1d_ref)
    w2_refs = (w2a_ref, w2b_ref)
    i = pl.program_id(0)

    @pl.when(i < NS)
    def _pool():
        sums = [jnp.sum(x_refs[k][...], axis=0, keepdims=True)
                for k in range(NSTREAM)]

        @pl.when(i == 0)
        def _init():
            for k in range(NSTREAM):
                acc8_ref[k:k + 1, :] = sums[k]

        @pl.when(i > 0)
        def _acc():
            for k in range(NSTREAM):
                acc8_ref[k:k + 1, :] += sums[k]

    @pl.when((i >= NS) & (i < NS + NH))
    def _mlp():
        j = i - NS

        @pl.when(j == 0)
        def _prep():
            a = acc8_ref[...]
            tot = a[0:B, :]
            for g in range(1, NSTREAM // B):
                tot = tot + a[g * B:(g + 1) * B, :]
            xp_ref[...] = tot * (1.0 / SEQ)

        xp = xp_ref[...]
        h1s = []
        for k in range(NSUB1):
            pre1 = lax.dot_general(xp, w1_refs[k][...],
                                   (((1,), (1,)), ((), ())),
                                   preferred_element_type=jnp.float32)
            h1s.append(
                jax.nn.gelu(pre1 + b1_ref[0, :, k * SUB1:(k + 1) * SUB1]))
        part = None
        for m in range(NSUB2):
            h1m = jnp.concatenate(h1s[2 * m:2 * m + 2], axis=1)
            p = lax.dot_general(h1m, w2_refs[m][...],
                                (((1,), (1,)), ((), ())),
                                preferred_element_type=jnp.float32)
            part = p if part is None else part + p

        @pl.when(j == 0)
        def _set():
            h2_ref[...] = part

        @pl.when(j > 0)
        def _add():
            h2_ref[...] += part

    @pl.when(i == NS + NH)
    def _final():
        h2 = jax.nn.gelu(h2_ref[...] + b2_ref[...])
        logits = lax.dot_general(h2, w3_ref[...], (((1,), (1,)), ((), ())),
                                 preferred_element_type=jnp.float32)
        logits = logits + b3_ref[...]
        logits_ref[...] = logits
        col = lax.broadcasted_iota(jnp.int32, (B, NUM_LAYERS), 1)
        maxv = jnp.max(logits, axis=1, keepdims=True)
        idx_ref[...] = jnp.min(
            jnp.where(logits == maxv, col, NUM_LAYERS), axis=1, keepdims=True)


def _x_spec(k):
    base = ((k % B) * SEQ + (k // B) * ROWS_PER_STREAM) // R_BLK
    return pl.BlockSpec(
        (R_BLK, D_MODEL),
        lambda i, b=base: (b + jnp.minimum(i, NS - 1), 0))


def _w1_spec(k):
    return pl.BlockSpec(
        (SUB1, D_MODEL),
        lambda i, k=k: (NSUB1 * jnp.clip(i - NS, 0, NH - 1) + k, 0))


def _w2_spec(k):
    return pl.BlockSpec(
        (HIDDEN, SUB2),
        lambda i, k=k: (0, NSUB2 * jnp.clip(i - NS, 0, NH - 1) + k))


def kernel(x, W1, b1, W2, b2, W3, b3):
    x2 = x.reshape(B * SEQ, D_MODEL)
    b1r = b1.reshape(NH, 1, H_BLK)
    b2r = b2.reshape(1, HIDDEN)
    b3r = b3.reshape(1, NUM_LAYERS)

    logits, idx = pl.pallas_call(
        _router_kernel,
        grid=(GRID,),
        in_specs=(
            [_x_spec(k) for k in range(NSTREAM)]
            + [_w1_spec(k) for k in range(NSUB1)]
            + [_w2_spec(k) for k in range(NSUB2)]
            + [pl.BlockSpec((1, 1, H_BLK),
                            lambda i: (jnp.clip(i - NS, 0, NH - 1), 0, 0)),
               pl.BlockSpec((1, HIDDEN), lambda i: (0, 0)),
               pl.BlockSpec((NUM_LAYERS, HIDDEN), lambda i: (0, 0)),
               pl.BlockSpec((1, NUM_LAYERS), lambda i: (0, 0))]
        ),
        out_specs=[
            pl.BlockSpec((B, NUM_LAYERS), lambda i: (0, 0)),
            pl.BlockSpec((B, 1), lambda i: (0, 0)),
        ],
        out_shape=[
            jax.ShapeDtypeStruct((B, NUM_LAYERS), jnp.float32),
            jax.ShapeDtypeStruct((B, 1), jnp.int32),
        ],
        scratch_shapes=[
            pltpu.VMEM((NSTREAM, D_MODEL), jnp.float32),
            pltpu.VMEM((B, D_MODEL), jnp.float32),
            pltpu.VMEM((B, HIDDEN), jnp.float32),
        ],
        compiler_params=pltpu.CompilerParams(
            dimension_semantics=("arbitrary",)),
    )(*([x2] * NSTREAM), W1, W1, W1, W1, W2, W2, b1r, b2r, W3, b3r)

    return (idx.reshape(B), logits)


# speedup vs baseline: 1.0054x; 1.0054x over previous
import jax
import jax.numpy as jnp
from jax import lax
from jax.experimental import pallas as pl
from jax.experimental.pallas import tpu as pltpu

B = 4
SEQ = 8192
D_MODEL = 4096
HIDDEN = 4096
NUM_LAYERS = 16

NSTREAM = 8
R_BLK = 64
ROWS_PER_STREAM = B * SEQ // NSTREAM
NS = ROWS_PER_STREAM // R_BLK

H_BLK = 512
NSUB = 4
SUB = H_BLK // NSUB
NH = HIDDEN // H_BLK
GRID = NS + NH


def _router_kernel(*refs):
    x_refs = refs[:NSTREAM]
    (w1a_ref, w1b_ref, w1c_ref, w1d_ref,
     w2a_ref, w2b_ref, w2c_ref, w2d_ref,
     b1_ref, b2_ref, w3_ref, b3_ref,
     logits_ref, idx_ref, acc8_ref, xp_ref, h2_ref) = refs[NSTREAM:]
    w1_refs = (w1a_ref, w1b_ref, w1c_ref, w1d_ref)
    w2_refs = (w2a_ref, w2b_ref, w2c_ref, w2d_ref)
    i = pl.program_id(0)

    @pl.when(i < NS)
    def _pool():
        sums = [jnp.sum(x_refs[k][...], axis=0, keepdims=True)
                for k in range(NSTREAM)]

        @pl.when(i == 0)
        def _init():
            for k in range(NSTREAM):
                acc8_ref[k:k + 1, :] = sums[k]

        @pl.when(i > 0)
        def _acc():
            for k in range(NSTREAM):
                acc8_ref[k:k + 1, :] += sums[k]

    @pl.when(i >= NS)
    def _mlp():
        j = i - NS

        @pl.when(j == 0)
        def _prep():
            a = acc8_ref[...]
            tot = a[0:B, :]
            for g in range(1, NSTREAM // B):
                tot = tot + a[g * B:(g + 1) * B, :]
            xp_ref[...] = tot * (1.0 / SEQ)

        xp = xp_ref[...]
        part = None
        for k in range(NSUB):
            pre1 = lax.dot_general(xp, w1_refs[k][...],
                                   (((1,), (1,)), ((), ())),
                                   preferred_element_type=jnp.float32)
            h1 = jax.nn.gelu(pre1 + b1_ref[0, :, k * SUB:(k + 1) * SUB])
            p = lax.dot_general(h1, w2_refs[k][...],
                                (((1,), (1,)), ((), ())),
                                preferred_element_type=jnp.float32)
            part = p if part is None else part + p

        @pl.when(j == 0)
        def _set():
            h2_ref[...] = part

        @pl.when(j > 0)
        def _add():
            h2_ref[...] += part

        @pl.when(j == NH - 1)
        def _final():
            h2 = jax.nn.gelu(h2_ref[...] + b2_ref[...])
            logits = lax.dot_general(h2, w3_ref[...],
                                     (((1,), (1,)), ((), ())),
                                     preferred_element_type=jnp.float32)
            logits = logits + b3_ref[...]
            logits_ref[...] = logits
            col = lax.broadcasted_iota(jnp.int32, (B, NUM_LAYERS), 1)
            maxv = jnp.max(logits, axis=1, keepdims=True)
            idx_ref[...] = jnp.min(
                jnp.where(logits == maxv, col, NUM_LAYERS),
                axis=1, keepdims=True)


def _x_spec(k):
    base = ((k % B) * SEQ + (k // B) * ROWS_PER_STREAM) // R_BLK
    return pl.BlockSpec(
        (R_BLK, D_MODEL),
        lambda i, b=base: (b + jnp.minimum(i, NS - 1), 0))


def _w1_spec(k):
    return pl.BlockSpec(
        (SUB, D_MODEL),
        lambda i, k=k: (NSUB * jnp.clip(i - NS, 0, NH - 1) + k, 0))


def _w2_spec(k):
    return pl.BlockSpec(
        (HIDDEN, SUB),
        lambda i, k=k: (0, NSUB * jnp.clip(i - NS, 0, NH - 1) + k))


def kernel(x, W1, b1, W2, b2, W3, b3):
    x2 = x.reshape(B * SEQ, D_MODEL)
    b1r = b1.reshape(NH, 1, H_BLK)
    b2r = b2.reshape(1, HIDDEN)
    b3r = b3.reshape(1, NUM_LAYERS)

    logits, idx = pl.pallas_call(
        _router_kernel,
        grid=(GRID,),
        in_specs=(
            [_x_spec(k) for k in range(NSTREAM)]
            + [_w1_spec(k) for k in range(NSUB)]
            + [_w2_spec(k) for k in range(NSUB)]
            + [pl.BlockSpec((1, 1, H_BLK),
                            lambda i: (jnp.clip(i - NS, 0, NH - 1), 0, 0)),
               pl.BlockSpec((1, HIDDEN), lambda i: (0, 0)),
               pl.BlockSpec((NUM_LAYERS, HIDDEN), lambda i: (0, 0)),
               pl.BlockSpec((1, NUM_LAYERS), lambda i: (0, 0))]
        ),
        out_specs=[
            pl.BlockSpec((B, NUM_LAYERS), lambda i: (0, 0)),
            pl.BlockSpec((B, 1), lambda i: (0, 0)),
        ],
        out_shape=[
            jax.ShapeDtypeStruct((B, NUM_LAYERS), jnp.float32),
            jax.ShapeDtypeStruct((B, 1), jnp.int32),
        ],
        scratch_shapes=[
            pltpu.VMEM((NSTREAM, D_MODEL), jnp.float32),
            pltpu.VMEM((B, D_MODEL), jnp.float32),
            pltpu.VMEM((B, HIDDEN), jnp.float32),
        ],
        compiler_params=pltpu.CompilerParams(
            dimension_semantics=("arbitrary",)),
    )(*([x2] * NSTREAM), W1, W1, W1, W1, W2, W2, W2, W2, b1r, b2r, W3, b3r)

    return (idx.reshape(B), logits)
